# Initial kernel scaffold; baseline (speedup 1.0000x reference)
import jax
import jax.numpy as jnp
from jax.experimental import pallas as pl


def kernel(x, rows, cols, vals):
    raise NotImplementedError("write your pallas kernel here")


# speedup vs baseline: 3.7960x; 3.7960x over previous
import jax
import jax.numpy as jnp
from jax import lax
from jax.experimental import pallas as pl
from jax.experimental.pallas import tpu as pltpu
from jax.experimental.pallas import tpu_sc as plsc

N = 16384
D = 256
LAYERS = 3
NT = 16
LANES = 16
SLABS = 4
SLAB_D = D // SLABS
BATCH = 128
NBUF = 2
HALVES = 2
ROWS_PER_TILE = N // NT


def _layer_body(x_in, idx4, rows_t, vals_t, out_hbm,
                shared, idx_v, rows_v, vals_v, gb0, gb1,
                gsem0, gsem1, ssem0, ssem1):
    nh = idx_v.shape[0]
    c = lax.axis_index("c")
    w = lax.axis_index("s")
    gbufs = (gb0, gb1)
    gsems = (gsem0, gsem1)
    ssems = (ssem0, ssem1)

    zero16 = jnp.zeros((LANES,), jnp.float32)

    for sp in range(2):
        s = 2 * c + sp

        @pl.loop(0, BATCH)
        def _zfill(i):
            for q in range(SLAB_D // LANES):
                gb0[i, pl.ds(q * LANES, LANES)] = zero16

        @pl.loop(0, ROWS_PER_TILE // BATCH)
        def _zero(k):
            pltpu.sync_copy(
                gb0, shared.at[pl.ds(w * ROWS_PER_TILE + k * BATCH, BATCH)])
        plsc.subcore_barrier()

        for h in range(HALVES):
            pltpu.sync_copy(idx4.at[s, w, pl.ds(h * nh, nh)], idx_v)
            pltpu.sync_copy(rows_t.at[w, pl.ds(h * nh, nh)], rows_v)
            pltpu.sync_copy(vals_t.at[w, pl.ds(h * nh, nh)], vals_v)

            for k in range(NBUF):
                pltpu.async_copy(x_in.at[idx_v.at[k]], gbufs[k], gsems[k])

            @pl.loop(0, nh, step=NBUF)
            def _main(b):
                for k in range(NBUF):
                    bi = b + k
                    gb = gbufs[k]
                    pltpu.make_async_copy(
                        x_in.at[idx_v.at[bi]], gb, gsems[k]).wait()

                    @pl.loop(0, BATCH // LANES)
                    def _mul(g):
                        v16 = vals_v[bi, pl.ds(g * LANES, LANES)]
                        for j in range(LANES):
                            splat = v16.at[jnp.full((LANES,), j, jnp.int32)
                                           ].get(mode="promise_in_bounds")
                            r = g * LANES + j
                            for q in range(SLAB_D // LANES):
                                sl = pl.ds(q * LANES, LANES)
                                gb[r, sl] = gb[r, sl] * splat

                    pltpu.async_copy(gb, shared.at[rows_v.at[bi]], ssems[k],
                                     add=True)

                    nxt = bi + NBUF

                    @pl.when(nxt < nh)
                    def _refill():
                        pltpu.make_async_copy(
                            gb, shared.at[rows_v.at[bi]], ssems[k]).wait()
                        pltpu.async_copy(x_in.at[idx_v.at[nxt]], gb, gsems[k])

            for k in range(NBUF):
                bi = nh - NBUF + k
                pltpu.make_async_copy(
                    gbufs[k], shared.at[rows_v.at[bi]], ssems[k]).wait()

        plsc.subcore_barrier()

        @pl.loop(0, ROWS_PER_TILE // BATCH)
        def _wb(k2):
            base = w * ROWS_PER_TILE + k2 * BATCH
            pltpu.sync_copy(shared.at[pl.ds(base, BATCH)],
                            out_hbm.at[pl.ds(s * N + base, BATCH)])
        plsc.subcore_barrier()


def _make_layer(nb):
    nh = nb // HALVES
    mesh = plsc.VectorSubcoreMesh(core_axis_name="c", subcore_axis_name="s")
    return pl.kernel(
        _layer_body,
        out_type=jax.ShapeDtypeStruct((SLABS * N, SLAB_D), jnp.float32),
        mesh=mesh,
        compiler_params=pltpu.CompilerParams(use_tc_tiling_on_sc=False),
        scratch_types=[
            pltpu.VMEM_SHARED((N, SLAB_D), jnp.float32),
            pltpu.VMEM((nh, BATCH), jnp.int32),
            pltpu.VMEM((nh, BATCH), jnp.int32),
            pltpu.VMEM((nh, BATCH), jnp.float32),
            pltpu.VMEM((BATCH, SLAB_D), jnp.float32),
            pltpu.VMEM((BATCH, SLAB_D), jnp.float32),
            pltpu.SemaphoreType.DMA,
            pltpu.SemaphoreType.DMA,
            pltpu.SemaphoreType.DMA,
            pltpu.SemaphoreType.DMA,
        ],
    )


def kernel(x, rows, cols, vals):
    nnz = rows.shape[0]
    group = NT * BATCH * HALVES
    nnz_pad = -(-nnz // group) * group
    nb = nnz_pad // (NT * BATCH)
    pad = nnz_pad - nnz

    cols_p = jnp.pad(cols, (0, pad))
    rows_p = jnp.pad(rows, (0, pad))
    vals_p = jnp.pad(vals, (0, pad))

    cols_t = cols_p.reshape(NT, nb, BATCH)
    idx4 = cols_t[None, ...] + (jnp.arange(SLABS, dtype=jnp.int32) * N)[
        :, None, None, None]
    rows_t = rows_p.reshape(NT, nb, BATCH)
    vals_t = vals_p.reshape(NT, nb, BATCH)

    xt = x.reshape(N, SLABS, SLAB_D).transpose(1, 0, 2).reshape(
        SLABS * N, SLAB_D)

    layer = _make_layer(nb)
    for _ in range(LAYERS):
        xt = layer(xt, idx4, rows_t, vals_t)

    return xt.reshape(SLABS, N, SLAB_D).transpose(1, 0, 2).reshape(N, D)


# speedup vs baseline: 7.9415x; 2.0921x over previous
import jax
import jax.numpy as jnp
from jax import lax
from jax.experimental import pallas as pl
from jax.experimental.pallas import tpu as pltpu
from jax.experimental.pallas import tpu_sc as plsc

N = 16384
D = 256
LAYERS = 3
NT = 16
LANES = 16
SLABS = 4
SLAB_D = D // SLABS
BATCH = 128
NBUF = 2
HALVES = 3
ROWS_PER_TILE = N // NT


def _layer_body(x_in, idx4, rows_t, vals_t, out_hbm,
                shared, idx_v, rows_v, vals_v, gb0, gb1, sb0, sb1,
                gsem0, gsem1, ssem0, ssem1):
    nh = idx_v.shape[0]
    c = lax.axis_index("c")
    w = lax.axis_index("s")
    gbufs = (gb0, gb1)
    sbufs = (sb0, sb1)
    gsems = (gsem0, gsem1)
    ssems = (ssem0, ssem1)

    zero16 = jnp.zeros((LANES,), jnp.float32)

    for sp in range(2):
        s = 2 * c + sp

        @pl.loop(0, BATCH)
        def _zfill(i):
            for q in range(SLAB_D // LANES):
                gb0[i, pl.ds(q * LANES, LANES)] = zero16

        @pl.loop(0, ROWS_PER_TILE // BATCH)
        def _zero(k):
            pltpu.sync_copy(
                gb0, shared.at[pl.ds(w * ROWS_PER_TILE + k * BATCH, BATCH)])
        plsc.subcore_barrier()

        for h in range(HALVES):
            pltpu.sync_copy(idx4.at[s, w, pl.ds(h * nh, nh)], idx_v)
            pltpu.sync_copy(rows_t.at[w, pl.ds(h * nh, nh)], rows_v)
            pltpu.sync_copy(vals_t.at[w, pl.ds(h * nh, nh)], vals_v)

            for k in range(NBUF):
                pltpu.async_copy(x_in.at[idx_v.at[k]], gbufs[k], gsems[k])

            @pl.loop(0, nh, step=NBUF)
            def _main(b):
                for k in range(NBUF):
                    bi = b + k
                    gb = gbufs[k]
                    sb = sbufs[k]
                    pltpu.make_async_copy(
                        x_in.at[idx_v.at[bi]], gb, gsems[k]).wait()

                    @pl.when(bi >= NBUF)
                    def _wait_prev_scatter():
                        pltpu.make_async_copy(
                            sb, shared.at[rows_v.at[bi]], ssems[k]).wait()

                    @pl.loop(0, BATCH // LANES)
                    def _mul(g):
                        v16 = vals_v[bi, pl.ds(g * LANES, LANES)]
                        for j in range(LANES):
                            splat = v16.at[jnp.full((LANES,), j, jnp.int32)
                                           ].get(mode="promise_in_bounds")
                            r = g * LANES + j
                            a = [gb[r, pl.ds(q * LANES, LANES)]
                                 for q in range(SLAB_D // LANES)]
                            for q in range(SLAB_D // LANES):
                                sb[r, pl.ds(q * LANES, LANES)] = a[q] * splat

                    pltpu.async_copy(sb, shared.at[rows_v.at[bi]], ssems[k],
                                     add=True)

                    nxt = bi + NBUF

                    @pl.when(nxt < nh)
                    def _refill():
                        pltpu.async_copy(x_in.at[idx_v.at[nxt]], gb, gsems[k])

            for k in range(NBUF):
                bi = nh - NBUF + k
                pltpu.make_async_copy(
                    sbufs[k], shared.at[rows_v.at[bi]], ssems[k]).wait()

        plsc.subcore_barrier()

        @pl.loop(0, ROWS_PER_TILE // BATCH)
        def _wb(k2):
            base = w * ROWS_PER_TILE + k2 * BATCH
            pltpu.sync_copy(shared.at[pl.ds(base, BATCH)],
                            out_hbm.at[pl.ds(s * N + base, BATCH)])
        plsc.subcore_barrier()


def _make_layer(nb):
    nh = nb // HALVES
    mesh = plsc.VectorSubcoreMesh(core_axis_name="c", subcore_axis_name="s")
    return pl.kernel(
        _layer_body,
        out_type=jax.ShapeDtypeStruct((SLABS * N, SLAB_D), jnp.float32),
        mesh=mesh,
        compiler_params=pltpu.CompilerParams(use_tc_tiling_on_sc=False),
        scratch_types=[
            pltpu.VMEM_SHARED((N, SLAB_D), jnp.float32),
            pltpu.VMEM((nh, BATCH), jnp.int32),
            pltpu.VMEM((nh, BATCH), jnp.int32),
            pltpu.VMEM((nh, BATCH), jnp.float32),
            pltpu.VMEM((BATCH, SLAB_D), jnp.float32),
            pltpu.VMEM((BATCH, SLAB_D), jnp.float32),
            pltpu.VMEM((BATCH, SLAB_D), jnp.float32),
            pltpu.VMEM((BATCH, SLAB_D), jnp.float32),
            pltpu.SemaphoreType.DMA,
            pltpu.SemaphoreType.DMA,
            pltpu.SemaphoreType.DMA,
            pltpu.SemaphoreType.DMA,
        ],
    )


def kernel(x, rows, cols, vals):
    nnz = rows.shape[0]
    group = NT * BATCH * HALVES
    nnz_pad = -(-nnz // group) * group
    nb = nnz_pad // (NT * BATCH)
    pad = nnz_pad - nnz

    cols_p = jnp.pad(cols, (0, pad))
    rows_p = jnp.pad(rows, (0, pad))
    vals_p = jnp.pad(vals, (0, pad))

    cols_t = cols_p.reshape(NT, nb, BATCH)
    idx4 = cols_t[None, ...] + (jnp.arange(SLABS, dtype=jnp.int32) * N)[
        :, None, None, None]
    rows_t = rows_p.reshape(NT, nb, BATCH)
    vals_t = vals_p.reshape(NT, nb, BATCH)

    xt = x.reshape(N, SLABS, SLAB_D).transpose(1, 0, 2).reshape(
        SLABS * N, SLAB_D)

    layer = _make_layer(nb)
    for _ in range(LAYERS):
        xt = layer(xt, idx4, rows_t, vals_t)

    return xt.reshape(SLABS, N, SLAB_D).transpose(1, 0, 2).reshape(N, D)
